# baseline (speedup 1.0000x reference)
import functools

import jax
import jax.numpy as jnp
from jax import lax
from jax.experimental import pallas as pl
from jax.experimental.pallas import tpu as pltpu
from jax.experimental.pallas import tpu_sc as plsc

N = 10000
E = 320000
D = 128
NW = 32
EPW = E // NW
LANES = 16
TILES = E // 128
NT_HI = 79
NT_LO = 78

_mesh = plsc.VectorSubcoreMesh(core_axis_name="c", subcore_axis_name="s")
_sc_params = pltpu.CompilerParams(needs_layout_passes=False)


@functools.partial(
    pl.kernel,
    mesh=_mesh,
    out_type=jax.ShapeDtypeStruct((NW, N), jnp.float32),
    compiler_params=_sc_params,
    scratch_types=[
        pltpu.VMEM((NT_HI, 2, 128), jnp.int32),
        pltpu.VMEM((N,), jnp.float32),
    ],
)
def _deg_kernel(ei_hbm, out_hbm, ei_v, acc_v):
    wid = lax.axis_index("c") * 16 + lax.axis_index("s")
    t0 = NT_LO * wid + jnp.minimum(wid, 4)

    zeros = jnp.zeros((LANES,), jnp.float32)

    def init(i, carry):
        acc_v[pl.ds(i * LANES, LANES)] = zeros
        return carry

    lax.fori_loop(0, N // LANES, init, 0, unroll=8)

    ones = jnp.ones((LANES,), jnp.float32)

    def run(nt):
        pltpu.sync_copy(ei_hbm.at[pl.ds(t0, nt)], ei_v.at[pl.ds(0, nt)])

        def body(t, carry):
            dvs = [ei_v[t, 1, pl.ds(k * LANES, LANES)] for k in range(8)]
            for dv in dvs:
                plsc.addupdate_scatter(acc_v, [dv], ones)
            return carry

        lax.fori_loop(0, nt, body, 0, unroll=4)

    @pl.when(wid < 4)
    def _():
        run(NT_HI)

    @pl.when(wid >= 4)
    def _():
        run(NT_LO)

    pltpu.sync_copy(acc_v, out_hbm.at[wid])


@functools.partial(
    pl.kernel,
    mesh=_mesh,
    out_type=jax.ShapeDtypeStruct((NW, N), jnp.float32),
    compiler_params=_sc_params,
    scratch_types=[
        pltpu.VMEM((NT_HI, 2, 128), jnp.int32),
        pltpu.VMEM((N,), jnp.float32),
        pltpu.VMEM((N,), jnp.float32),
    ],
)
def _agg_kernel(ei_hbm, s_hbm, out_hbm, ei_v, s_v, acc_v):
    wid = lax.axis_index("c") * 16 + lax.axis_index("s")
    t0 = NT_LO * wid + jnp.minimum(wid, 4)
    pltpu.sync_copy(s_hbm, s_v)

    zeros = jnp.zeros((LANES,), jnp.float32)

    def init(i, carry):
        acc_v[pl.ds(i * LANES, LANES)] = zeros
        return carry

    lax.fori_loop(0, N // LANES, init, 0, unroll=8)

    def run(nt):
        pltpu.sync_copy(ei_hbm.at[pl.ds(t0, nt)], ei_v.at[pl.ds(0, nt)])

        def body(t, carry):
            svs = [
                plsc.load_gather(s_v, [ei_v[t, 0, pl.ds(k * LANES, LANES)]])
                for k in range(8)
            ]
            for k in range(8):
                plsc.addupdate_scatter(
                    acc_v, [ei_v[t, 1, pl.ds(k * LANES, LANES)]], svs[k])
            return carry

        lax.fori_loop(0, nt, body, 0, unroll=8)

    @pl.when(wid < 4)
    def _():
        run(NT_HI)

    @pl.when(wid >= 4)
    def _():
        run(NT_LO)

    pltpu.sync_copy(acc_v, out_hbm.at[wid])


def _tc0_body(x_ref, wt_ref, xw_ref):
    xw_ref[...] = lax.dot_general(
        wt_ref[...], x_ref[...], (((1,), (1,)), ((), ())),
        preferred_element_type=jnp.float32)


def _tc1_body(xw_ref, degp_ref, s_ref, dinv_ref, s1_ref):
    deg = jnp.sum(degp_ref[...], axis=0, keepdims=True) + 1.0
    dinv = lax.rsqrt(deg)
    dinv_ref[...] = dinv
    s = xw_ref[...] * dinv
    s_ref[...] = s
    s1_ref[...] = s.reshape(N)


def _tc2_body(accp_ref, s_ref, dinv_ref, b_ref, o_ref):
    tot = jnp.sum(accp_ref[...], axis=0, keepdims=True) + s_ref[...]
    o_ref[...] = jnp.maximum(dinv_ref[...] * tot + b_ref[...], 0.0)


def kernel(x, edge_index, W, b):
    ei = edge_index.astype(jnp.int32).reshape(2, TILES, 128).transpose(1, 0, 2)
    wt = W.reshape(1, D)
    b2 = b.reshape(1, 1)

    xw_row = pl.pallas_call(
        _tc0_body,
        out_shape=jax.ShapeDtypeStruct((1, N), jnp.float32),
    )(x, wt)

    degp = _deg_kernel(ei)

    s_row, dinv_row, s1d = pl.pallas_call(
        _tc1_body,
        out_shape=[
            jax.ShapeDtypeStruct((1, N), jnp.float32),
            jax.ShapeDtypeStruct((1, N), jnp.float32),
            jax.ShapeDtypeStruct((N,), jnp.float32),
        ],
    )(xw_row, degp)

    accp = _agg_kernel(ei, s1d)

    out_row = pl.pallas_call(
        _tc2_body,
        out_shape=jax.ShapeDtypeStruct((1, N), jnp.float32),
    )(accp, s_row, dinv_row, b2)

    return out_row.reshape(N, 1)


# speedup vs baseline: 1.0486x; 1.0486x over previous
import functools

import jax
import jax.numpy as jnp
from jax import lax
from jax.experimental import pallas as pl
from jax.experimental.pallas import tpu as pltpu
from jax.experimental.pallas import tpu_sc as plsc

N = 10000
E = 320000
D = 128
NW = 32
EPW = E // NW
LANES = 16
TILES = E // 128
NT_HI = 79
NT_LO = 78

_mesh = plsc.VectorSubcoreMesh(core_axis_name="c", subcore_axis_name="s")
_sc_params = pltpu.CompilerParams(needs_layout_passes=False)


@functools.partial(
    pl.kernel,
    mesh=_mesh,
    out_type=jax.ShapeDtypeStruct((NW, N), jnp.float32),
    compiler_params=_sc_params,
    scratch_types=[
        pltpu.VMEM((NT_HI, 2, 128), jnp.int32),
        pltpu.VMEM((N,), jnp.float32),
    ],
)
def _deg_kernel(ei_hbm, out_hbm, ei_v, acc_v):
    wid = lax.axis_index("c") * 16 + lax.axis_index("s")
    t0 = NT_LO * wid + jnp.minimum(wid, 4)

    zeros = jnp.zeros((LANES,), jnp.float32)

    def init(i, carry):
        acc_v[pl.ds(i * LANES, LANES)] = zeros
        return carry

    lax.fori_loop(0, N // LANES, init, 0, unroll=8)

    ones = jnp.ones((LANES,), jnp.float32)

    def run(nt):
        pltpu.sync_copy(ei_hbm.at[pl.ds(t0, nt)], ei_v.at[pl.ds(0, nt)])

        def body(t, carry):
            dvs = [ei_v[t, 1, pl.ds(k * LANES, LANES)] for k in range(8)]
            for dv in dvs:
                plsc.addupdate_scatter(acc_v, [dv], ones)
            return carry

        lax.fori_loop(0, nt, body, 0, unroll=2)

    @pl.when(wid < 4)
    def _():
        run(NT_HI)

    @pl.when(wid >= 4)
    def _():
        run(NT_LO)

    pltpu.sync_copy(acc_v, out_hbm.at[wid])


@functools.partial(
    pl.kernel,
    mesh=_mesh,
    out_type=jax.ShapeDtypeStruct((NW, N), jnp.float32),
    compiler_params=_sc_params,
    scratch_types=[
        pltpu.VMEM((NT_HI, 2, 128), jnp.int32),
        pltpu.VMEM((N,), jnp.float32),
        pltpu.VMEM((N,), jnp.float32),
    ],
)
def _agg_kernel(ei_hbm, s_hbm, out_hbm, ei_v, s_v, acc_v):
    wid = lax.axis_index("c") * 16 + lax.axis_index("s")
    t0 = NT_LO * wid + jnp.minimum(wid, 4)
    pltpu.sync_copy(s_hbm, s_v)

    zeros = jnp.zeros((LANES,), jnp.float32)

    def init(i, carry):
        acc_v[pl.ds(i * LANES, LANES)] = zeros
        return carry

    lax.fori_loop(0, N // LANES, init, 0, unroll=8)

    def run(nt):
        pltpu.sync_copy(ei_hbm.at[pl.ds(t0, nt)], ei_v.at[pl.ds(0, nt)])

        def body(t, carry):
            svs = [
                plsc.load_gather(s_v, [ei_v[t, 0, pl.ds(k * LANES, LANES)]])
                for k in range(8)
            ]
            for k in range(8):
                plsc.addupdate_scatter(
                    acc_v, [ei_v[t, 1, pl.ds(k * LANES, LANES)]], svs[k])
            return carry

        lax.fori_loop(0, nt, body, 0, unroll=4)

    @pl.when(wid < 4)
    def _():
        run(NT_HI)

    @pl.when(wid >= 4)
    def _():
        run(NT_LO)

    pltpu.sync_copy(acc_v, out_hbm.at[wid])


def _tc0_body(x_ref, wt_ref, xw_ref):
    xw_ref[...] = lax.dot_general(
        wt_ref[...], x_ref[...], (((1,), (1,)), ((), ())),
        preferred_element_type=jnp.float32)


def _tc1_body(xw_ref, degp_ref, s_ref, dinv_ref, s1_ref):
    deg = jnp.sum(degp_ref[...], axis=0, keepdims=True) + 1.0
    dinv = lax.rsqrt(deg)
    dinv_ref[...] = dinv
    s = xw_ref[...] * dinv
    s_ref[...] = s
    s1_ref[...] = s.reshape(N)


def _tc2_body(accp_ref, s_ref, dinv_ref, b_ref, o_ref):
    tot = jnp.sum(accp_ref[...], axis=0, keepdims=True) + s_ref[...]
    o_ref[...] = jnp.maximum(dinv_ref[...] * tot + b_ref[...], 0.0)


def kernel(x, edge_index, W, b):
    ei = edge_index.astype(jnp.int32).reshape(2, TILES, 128).transpose(1, 0, 2)
    wt = W.reshape(1, D)
    b2 = b.reshape(1, 1)

    xw_row = pl.pallas_call(
        _tc0_body,
        out_shape=jax.ShapeDtypeStruct((1, N), jnp.float32),
    )(x, wt)

    degp = _deg_kernel(ei)

    s_row, dinv_row, s1d = pl.pallas_call(
        _tc1_body,
        out_shape=[
            jax.ShapeDtypeStruct((1, N), jnp.float32),
            jax.ShapeDtypeStruct((1, N), jnp.float32),
            jax.ShapeDtypeStruct((N,), jnp.float32),
        ],
    )(xw_row, degp)

    accp = _agg_kernel(ei, s1d)

    out_row = pl.pallas_call(
        _tc2_body,
        out_shape=jax.ShapeDtypeStruct((1, N), jnp.float32),
    )(accp, s_row, dinv_row, b2)

    return out_row.reshape(N, 1)


# speedup vs baseline: 1.0772x; 1.0273x over previous
import functools

import jax
import jax.numpy as jnp
from jax import lax
from jax.experimental import pallas as pl
from jax.experimental.pallas import tpu as pltpu
from jax.experimental.pallas import tpu_sc as plsc

N = 10000
E = 320000
D = 128
NW = 32
EPW = E // NW
LANES = 16
TILES = E // 128
NT_HI = 79
NT_LO = 78

_mesh = plsc.VectorSubcoreMesh(core_axis_name="c", subcore_axis_name="s")
_sc_params = pltpu.CompilerParams(needs_layout_passes=False)


@functools.partial(
    pl.kernel,
    mesh=_mesh,
    out_type=jax.ShapeDtypeStruct((NW, N), jnp.float32),
    compiler_params=_sc_params,
    scratch_types=[
        pltpu.VMEM((NT_HI, 2, 128), jnp.int32),
        pltpu.VMEM((N,), jnp.float32),
        pltpu.SemaphoreType.DMA,
    ],
)
def _deg_kernel(ei_hbm, out_hbm, ei_v, acc_v, sem):
    wid = lax.axis_index("c") * 16 + lax.axis_index("s")
    t0 = NT_LO * wid + jnp.minimum(wid, 4)

    zeros = jnp.zeros((LANES,), jnp.float32)
    ones = jnp.ones((LANES,), jnp.float32)

    def init(i, carry):
        acc_v[pl.ds(i * LANES, LANES)] = zeros
        return carry

    def run(nt):
        cp = pltpu.async_copy(
            ei_hbm.at[pl.ds(t0, nt)], ei_v.at[pl.ds(0, nt)], sem)
        lax.fori_loop(0, N // LANES, init, 0, unroll=8)
        cp.wait()

        def body(t, carry):
            dvs = [ei_v[t, 1, pl.ds(k * LANES, LANES)] for k in range(8)]
            for dv in dvs:
                plsc.addupdate_scatter(acc_v, [dv], ones)
            return carry

        lax.fori_loop(0, nt, body, 0, unroll=2)

    @pl.when(wid < 4)
    def _():
        run(NT_HI)

    @pl.when(wid >= 4)
    def _():
        run(NT_LO)

    pltpu.sync_copy(acc_v, out_hbm.at[wid])


@functools.partial(
    pl.kernel,
    mesh=_mesh,
    out_type=jax.ShapeDtypeStruct((NW, N), jnp.float32),
    compiler_params=_sc_params,
    scratch_types=[
        pltpu.VMEM((NT_HI, 2, 128), jnp.int32),
        pltpu.VMEM((N,), jnp.float32),
        pltpu.VMEM((N,), jnp.float32),
        pltpu.SemaphoreType.DMA,
        pltpu.SemaphoreType.DMA,
    ],
)
def _agg_kernel(ei_hbm, s_hbm, out_hbm, ei_v, s_v, acc_v, sem_s, sem_e):
    wid = lax.axis_index("c") * 16 + lax.axis_index("s")
    t0 = NT_LO * wid + jnp.minimum(wid, 4)

    zeros = jnp.zeros((LANES,), jnp.float32)

    def init(i, carry):
        acc_v[pl.ds(i * LANES, LANES)] = zeros
        return carry

    def run(nt):
        cp_s = pltpu.async_copy(s_hbm, s_v, sem_s)
        cp_e = pltpu.async_copy(
            ei_hbm.at[pl.ds(t0, nt)], ei_v.at[pl.ds(0, nt)], sem_e)
        lax.fori_loop(0, N // LANES, init, 0, unroll=8)
        cp_s.wait()
        cp_e.wait()

        def body(t, carry):
            svs = [
                plsc.load_gather(s_v, [ei_v[t, 0, pl.ds(k * LANES, LANES)]])
                for k in range(8)
            ]
            for k in range(8):
                plsc.addupdate_scatter(
                    acc_v, [ei_v[t, 1, pl.ds(k * LANES, LANES)]], svs[k])
            return carry

        lax.fori_loop(0, nt, body, 0, unroll=4)

    @pl.when(wid < 4)
    def _():
        run(NT_HI)

    @pl.when(wid >= 4)
    def _():
        run(NT_LO)

    pltpu.sync_copy(acc_v, out_hbm.at[wid])


def _tc0_body(x_ref, wt_ref, xw_ref):
    xw_ref[...] = lax.dot_general(
        wt_ref[...], x_ref[...], (((1,), (1,)), ((), ())),
        preferred_element_type=jnp.float32)


def _tc1_body(xw_ref, degp_ref, s_ref, dinv_ref, s1_ref):
    deg = jnp.sum(degp_ref[...], axis=0, keepdims=True) + 1.0
    dinv = lax.rsqrt(deg)
    dinv_ref[...] = dinv
    s = xw_ref[...] * dinv
    s_ref[...] = s
    s1_ref[...] = s.reshape(N)


def _tc2_body(accp_ref, s_ref, dinv_ref, b_ref, o_ref):
    tot = jnp.sum(accp_ref[...], axis=0, keepdims=True) + s_ref[...]
    o_ref[...] = jnp.maximum(dinv_ref[...] * tot + b_ref[...], 0.0)


def kernel(x, edge_index, W, b):
    ei = edge_index.astype(jnp.int32).reshape(2, TILES, 128).transpose(1, 0, 2)
    wt = W.reshape(1, D)
    b2 = b.reshape(1, 1)

    xw_row = pl.pallas_call(
        _tc0_body,
        out_shape=jax.ShapeDtypeStruct((1, N), jnp.float32),
    )(x, wt)

    degp = _deg_kernel(ei)

    s_row, dinv_row, s1d = pl.pallas_call(
        _tc1_body,
        out_shape=[
            jax.ShapeDtypeStruct((1, N), jnp.float32),
            jax.ShapeDtypeStruct((1, N), jnp.float32),
            jax.ShapeDtypeStruct((N,), jnp.float32),
        ],
    )(xw_row, degp)

    accp = _agg_kernel(ei, s1d)

    out_row = pl.pallas_call(
        _tc2_body,
        out_shape=jax.ShapeDtypeStruct((1, N), jnp.float32),
    )(accp, s_row, dinv_row, b2)

    return out_row.reshape(N, 1)
